# baseline (speedup 1.0000x reference)
import jax
import jax.numpy as jnp
from jax.experimental import pallas as pl
from jax.experimental.pallas import tpu as pltpu
from jax.experimental.pallas import tpu_sc as plsc

_CHUNK = 6144


def kernel(inputs, pos_table):
    B, S, D = inputs.shape
    N = B * S * D
    TD = S * D
    n_tab = TD // _CHUNK
    x1 = inputs.reshape(N)
    t1 = pos_table.reshape(TD)
    mesh = plsc.VectorSubcoreMesh(core_axis_name="c", subcore_axis_name="s")

    @pl.kernel(out_type=jax.ShapeDtypeStruct((N,), inputs.dtype), mesh=mesh)
    def sc_add(x_hbm, t_hbm, o_hbm):
        def body(x_vmem, t_vmem, o_vmem):
            @plsc.parallel_loop(0, _CHUNK, step=16, unroll=8)
            def _(c):
                o_vmem[pl.ds(c, 16)] = x_vmem[pl.ds(c, 16)] + t_vmem[pl.ds(c, 16)]

        pltpu.emit_pipeline(
            body,
            grid=(N // _CHUNK,),
            in_specs=[
                pl.BlockSpec((_CHUNK,), lambda i: (i,)),
                pl.BlockSpec((_CHUNK,), lambda i: (jax.lax.rem(i, n_tab),)),
            ],
            out_specs=[pl.BlockSpec((_CHUNK,), lambda i: (i,))],
            core_axis_name=("c", "s"),
            dimension_semantics=(pltpu.PARALLEL,),
        )(x_hbm, t_hbm, o_hbm)

    return sc_add(x1, t1).reshape(B, S, D)


# speedup vs baseline: 5.1530x; 5.1530x over previous
import jax
import jax.numpy as jnp
from jax.experimental import pallas as pl


def _add_body(x_ref, t_ref, o_ref):
    o_ref[...] = x_ref[...] + t_ref[...][None, :, :]


def kernel(inputs, pos_table):
    B, S, D = inputs.shape
    BS = 512
    return pl.pallas_call(
        _add_body,
        grid=(S // BS,),
        in_specs=[
            pl.BlockSpec((B, BS, D), lambda i: (0, i, 0)),
            pl.BlockSpec((BS, D), lambda i: (i, 0)),
        ],
        out_specs=pl.BlockSpec((B, BS, D), lambda i: (0, i, 0)),
        out_shape=jax.ShapeDtypeStruct((B, S, D), inputs.dtype),
    )(inputs, pos_table)
